# baseline (speedup 1.0000x reference)
import math

import jax
import jax.numpy as jnp
from jax.experimental import pallas as pl
from jax.experimental.pallas import tpu as pltpu

_V = 8192
_SMOOTH = 0.1
_CONF = 1.0 - _SMOOTH
_PAD = 1
_EPS = _SMOOTH / (_V - 1)
_C = _CONF * math.log(_CONF) + _SMOOTH * math.log(_EPS)
_BLK = 512


def _loss_kernel(t_ref, x_ref, sum_ref, cnt_ref):
    i = pl.program_id(0)

    @pl.when(i == 0)
    def _():
        sum_ref[0, 0] = 0.0
        cnt_ref[0, 0] = 0.0

    xb = x_ref[...]
    t = t_ref[0, 0, :]
    m = jnp.max(xb, axis=1, keepdims=True)
    e = jnp.exp(xb - m)
    ones = jnp.ones((_V, 1), jnp.float32)
    s = jax.lax.dot_general(e, ones, (((1,), (0,)), ((), ())),
                            preferred_element_type=jnp.float32)[:, 0]
    idx = jax.lax.broadcasted_iota(jnp.int32, xb.shape, 1)
    w = jnp.where(idx == t[:, None], _CONF, _EPS)
    wsum = jnp.sum(xb * w, axis=1)
    keep = (t != _PAD).astype(jnp.float32)
    per = _C + m[:, 0] + jnp.log(s) - wsum
    sum_ref[0, 0] += jnp.sum(per * keep)
    cnt_ref[0, 0] += jnp.sum(keep)


@jax.jit
def kernel(x, target):
    xf = x.reshape(-1, _V)
    n = xf.shape[0]
    nblk = n // _BLK
    t = target.reshape(-1).astype(jnp.int32).reshape(nblk, 1, _BLK)
    loss_sum, cnt = pl.pallas_call(
        _loss_kernel,
        grid=(nblk,),
        in_specs=[
            pl.BlockSpec((1, 1, _BLK), lambda i: (i, 0, 0)),
            pl.BlockSpec((_BLK, _V), lambda i: (i, 0)),
        ],
        out_specs=[
            pl.BlockSpec(memory_space=pltpu.SMEM),
            pl.BlockSpec(memory_space=pltpu.SMEM),
        ],
        out_shape=[
            jax.ShapeDtypeStruct((1, 1), jnp.float32),
            jax.ShapeDtypeStruct((1, 1), jnp.float32),
        ],
    )(t, xf)
    return loss_sum[0, 0] / cnt[0, 0]


# speedup vs baseline: 1.0078x; 1.0078x over previous
import math

import jax
import jax.numpy as jnp
from jax.experimental import pallas as pl
from jax.experimental.pallas import tpu as pltpu

_V = 8192
_SMOOTH = 0.1
_CONF = 1.0 - _SMOOTH
_PAD = 1
_EPS = _SMOOTH / (_V - 1)
_C = _CONF * math.log(_CONF) + _SMOOTH * math.log(_EPS)
_BLK = 512


def _loss_kernel(t_ref, x_ref, sum_ref, cnt_ref):
    i = pl.program_id(0)

    @pl.when(i == 0)
    def _():
        sum_ref[0, 0] = 0.0
        cnt_ref[0, 0] = 0.0

    xb = x_ref[...]
    t = t_ref[0, 0, :]
    m = jnp.max(xb, axis=1, keepdims=True)
    s = jnp.sum(jnp.exp(xb - m), axis=1)
    idx = jax.lax.broadcasted_iota(jnp.int32, xb.shape, 1)
    w = jnp.where(idx == t[:, None], _CONF, _EPS)
    wsum = jnp.sum(xb * w, axis=1)
    keep = (t != _PAD).astype(jnp.float32)
    per = _C + m[:, 0] + jnp.log(s) - wsum
    sum_ref[0, 0] += jnp.sum(per * keep)
    cnt_ref[0, 0] += jnp.sum(keep)


@jax.jit
def kernel(x, target):
    xf = x.reshape(-1, _V)
    n = xf.shape[0]
    nblk = n // _BLK
    t = target.reshape(-1).astype(jnp.int32).reshape(nblk, 1, _BLK)
    loss_sum, cnt = pl.pallas_call(
        _loss_kernel,
        grid=(nblk,),
        in_specs=[
            pl.BlockSpec((1, 1, _BLK), lambda i: (i, 0, 0)),
            pl.BlockSpec((_BLK, _V), lambda i: (i, 0)),
        ],
        out_specs=[
            pl.BlockSpec(memory_space=pltpu.SMEM),
            pl.BlockSpec(memory_space=pltpu.SMEM),
        ],
        out_shape=[
            jax.ShapeDtypeStruct((1, 1), jnp.float32),
            jax.ShapeDtypeStruct((1, 1), jnp.float32),
        ],
    )(t, xf)
    return loss_sum[0, 0] / cnt[0, 0]
